# Initial kernel scaffold; baseline (speedup 1.0000x reference)
import jax
import jax.numpy as jnp
from jax.experimental import pallas as pl


def kernel(embeddings, cluster_assignments, generation, consumption, positions, current_hour, W1, b1, W2, b2, W3, b3, We1, be1, We2, be2, Wp1, bp1, Wp2, bp2):
    raise NotImplementedError("write your pallas kernel here")


# speedup vs baseline: 5108.3986x; 5108.3986x over previous
import jax
import jax.numpy as jnp
from jax import lax
from jax.experimental import pallas as pl
from jax.experimental.pallas import tpu as pltpu

_N = 128
_R = 16
_NB = _N // _R
_NCLUST = 8
_F32 = jnp.float32

_HI = lax.Precision.HIGHEST


def _mm(a, b, precision=None):
    return lax.dot_general(a, b, dimension_numbers=(((1,), (0,)), ((), ())),
                           preferred_element_type=_F32, precision=precision)


def _dense_kernel(emb_ref, px_ref, py_ref, pxb_ref, pyb_ref,
                  W1aT_ref, W1bT_ref, wd_ref, g_ref,
                  W2T_ref, b2_ref, W3T_ref, b3_ref,
                  We1T_ref, be1_ref, We2T_ref, be2_ref,
                  Wp1T_ref, bp1_ref, Wp2T_ref, bp2_ref,
                  pred_ref, eff_ref, pri_ref, A_ref, B_ref):
    b = pl.program_id(0)

    @pl.when(b == 0)
    def _():
        A_ref[...] = _mm(emb_ref[...], W1aT_ref[...]) + g_ref[...]
        B_ref[...] = _mm(emb_ref[...], W1bT_ref[...])
        hp = jnp.maximum(_mm(emb_ref[...], Wp1T_ref[...]) + bp1_ref[...], 0.0)
        pri_ref[...] = jax.nn.sigmoid(_mm(hp, Wp2T_ref[...]) + bp2_ref[...])

    pxi = pxb_ref[...].reshape(_R, 1, 1)
    pyi = pyb_ref[...].reshape(_R, 1, 1)
    pxj = px_ref[...].reshape(1, _N, 1)
    pyj = py_ref[...].reshape(1, _N, 1)
    dx = jnp.broadcast_to(pxi, (_R, _N, 1)) - jnp.broadcast_to(pxj, (_R, _N, 1))
    dy = jnp.broadcast_to(pyi, (_R, _N, 1)) - jnp.broadcast_to(pyj, (_R, _N, 1))
    dist = jnp.sqrt(dx * dx + dy * dy).reshape(_R * _N, 1)

    he = jnp.maximum(_mm(dist * (1.0 / 1000.0), We1T_ref[...]) + be1_ref[...],
                     0.0)
    se = jax.nn.sigmoid(_mm(he, We2T_ref[...]) + be2_ref[...])
    eff_ref[...] = (0.85 + 0.13 * se).reshape(_R, _N, 1)

    a3 = A_ref[pl.ds(b * _R, _R), :].reshape(_R, 1, _N)
    ab = jnp.broadcast_to(a3, (_R, _N, _N)).reshape(_R * _N, _N)
    bb = jnp.broadcast_to(B_ref[...].reshape(1, _N, _N),
                          (_R, _N, _N)).reshape(_R * _N, _N)
    h1 = jnp.maximum(ab + bb + dist * wd_ref[...], 0.0)
    h2 = jnp.maximum(_mm(h1, W2T_ref[...]) + b2_ref[...], 0.0)
    pr = _mm(h2, W3T_ref[...]) + b3_ref[...]
    pred_ref[...] = jax.nn.softplus(pr).reshape(_R, _N, 1)


def _run_dense(emb, px, py, W1aT, W1bT, wd_row, g_row, W2T, b2r,
               W3T, b3r, We1T, be1r, We2T, be2r, Wp1T, bp1r, Wp2T, bp2r):
    full = lambda shp: pl.BlockSpec(shp, lambda b: tuple(0 for _ in shp))
    in_specs = [
        full((_N, _N)),
        full((_N, 1)), full((_N, 1)),
        pl.BlockSpec((_R, 1), lambda b: (b, 0)),
        pl.BlockSpec((_R, 1), lambda b: (b, 0)),
        full(W1aT.shape), full(W1bT.shape), full(wd_row.shape),
        full(g_row.shape), full(W2T.shape), full(b2r.shape),
        full(W3T.shape), full(b3r.shape), full(We1T.shape), full(be1r.shape),
        full(We2T.shape), full(be2r.shape), full(Wp1T.shape), full(bp1r.shape),
        full(Wp2T.shape), full(bp2r.shape),
    ]
    out_specs = [
        pl.BlockSpec((_R, _N, 1), lambda b: (b, 0, 0)),
        pl.BlockSpec((_R, _N, 1), lambda b: (b, 0, 0)),
        full((_N, 1)),
    ]
    out_shape = [
        jax.ShapeDtypeStruct((_N, _N, 1), _F32),
        jax.ShapeDtypeStruct((_N, _N, 1), _F32),
        jax.ShapeDtypeStruct((_N, 1), _F32),
    ]
    pred3, eff3, pri = pl.pallas_call(
        _dense_kernel,
        grid=(_NB,),
        in_specs=in_specs,
        out_specs=out_specs,
        out_shape=out_shape,
        scratch_shapes=[pltpu.VMEM((_N, _N), _F32), pltpu.VMEM((_N, _N), _F32)],
    )(emb, px, py, px, py, W1aT, W1bT, wd_row, g_row, W2T, b2r, W3T,
      b3r, We1T, be1r, We2T, be2r, Wp1T, bp1r, Wp2T, bp2r)
    return pred3.reshape(_N, _N), eff3.reshape(_N, _N), pri


def _greedy_kernel(pred_ref, eff_ref, pri_row_ref, pri_col_ref,
                   ca_row_ref, ca_col_ref, n0_row_ref, n0_col_ref,
                   sharing_ref, effmat_ref, net_ref, esent_ref, erecv_ref,
                   total_ref, sharingP_ref, effPd_ref):
    pred = pred_ref[...]
    eff = eff_ref[...]
    prir = pri_row_ref[...]
    pric = pri_col_ref[...]
    car = ca_row_ref[...]
    cac = ca_col_ref[...]
    n0r = n0_row_ref[...]
    n0c = n0_col_ref[...]

    iota_sub = lax.broadcasted_iota(jnp.int32, (_N, _N), 0).astype(_F32)
    iota_lan = lax.broadcasted_iota(jnp.int32, (_N, _N), 1).astype(_F32)

    isdef_r = n0r < 0.0
    isdef_c = n0c < 0.0
    issur_r = n0r > 0.0
    issur_c = n0c > 0.0

    one = jnp.float32(1.0)
    zero = jnp.float32(0.0)

    beforeA = (car < cac) | ((car == cac) & (
        (prir > pric) | ((prir == pric) & (iota_lan < iota_sub))))
    beforeB = (cac < car) | ((cac == car) & (
        (pric > prir) | ((pric == prir) & (iota_sub < iota_lan))))

    rank_d_col = jnp.sum(jnp.where(beforeA & isdef_r, one, zero),
                         axis=1, keepdims=True)
    rank_s_col = jnp.sum(jnp.where(beforeA & issur_r, one, zero),
                         axis=1, keepdims=True)
    rank_d_row = jnp.sum(jnp.where(beforeB & isdef_c, one, zero),
                         axis=0, keepdims=True)
    rank_s_row = jnp.sum(jnp.where(beforeB & issur_c, one, zero),
                         axis=0, keepdims=True)

    D2 = jnp.where(isdef_c & (rank_d_col == iota_lan), one, zero)
    D2T = jnp.where(isdef_r & (rank_d_row == iota_sub), one, zero)
    S2 = jnp.where(issur_c & (rank_s_col == iota_lan), one, zero)
    S2T = jnp.where(issur_r & (rank_s_row == iota_sub), one, zero)

    segr = _mm(car, D2, _HI)
    segc = _mm(D2T, cac, _HI)
    dval_r = jnp.sum(D2, axis=0, keepdims=True) > 0.5
    needed0 = -_mm(n0r, D2, _HI)
    Apack_c = _mm(S2T, n0c, _HI)

    PredP = _mm(_mm(S2T, pred, _HI), D2, _HI)
    EffP = _mm(_mm(S2T, eff, _HI), D2, _HI)

    io8 = lax.broadcasted_iota(jnp.int32, (_NCLUST, _N), 0).astype(_F32)
    cnt8 = jnp.sum(jnp.where((car == io8) & issur_r, one, zero),
                   axis=1, keepdims=True)
    start8 = jnp.sum(jnp.where((car < io8) & issur_r, one, zero),
                     axis=1, keepdims=True)
    segOH = jnp.where(segr == io8, one, zero)
    scount_r = jnp.sum(segOH * cnt8, axis=0, keepdims=True)
    sstart_r = jnp.sum(segOH * start8, axis=0, keepdims=True)

    MT = jnp.where((iota_sub <= iota_lan) & (segc == segr), one, zero)

    nsteps = jnp.max(cnt8).astype(jnp.int32)

    sharingP_ref[...] = jnp.zeros((_N, _N), _F32)
    effPd_ref[...] = jnp.zeros((_N, _N), _F32)

    def body(i, needed):
        fi = i.astype(_F32)
        rowsel = sstart_r + fi
        OHr = iota_sub == rowsel
        predsel = jnp.sum(jnp.where(OHr, PredP, zero), axis=0, keepdims=True)
        effsel = jnp.sum(jnp.where(OHr, EffP, zero), axis=0, keepdims=True)
        Asel = jnp.sum(jnp.where(OHr, Apack_c, zero), axis=0, keepdims=True)
        gate0 = dval_r & (fi < scount_r) & (needed > 0.0)
        cap = jnp.where(gate0, jnp.minimum(needed, predsel), zero)
        C = _mm(cap, MT, _HI)
        Cex = C - cap
        upd = gate0 & (Asel - Cex > 0.0)
        actual = jnp.where(
            upd, jnp.minimum(Asel, C) - jnp.minimum(Asel, Cex), zero)
        delivered = actual * effsel
        mask2 = OHr & upd
        sharingP_ref[...] += jnp.where(mask2, actual, zero)
        effPd_ref[...] += jnp.where(mask2, effsel - one, zero)
        return needed - jnp.where(upd, delivered, zero)

    needed_fin = lax.fori_loop(0, nsteps, body, needed0)

    sharing = _mm(_mm(S2, sharingP_ref[...], _HI), D2T, _HI)
    effmat = 1.0 + _mm(_mm(S2, effPd_ref[...], _HI), D2T, _HI)
    neededU = _mm(needed_fin, D2T, _HI)
    net_out = jnp.where(isdef_r, -neededU, n0r)

    sharing_ref[...] = sharing
    effmat_ref[...] = effmat
    net_ref[...] = net_out
    esent_ref[...] = jnp.sum(sharing, axis=1, keepdims=True)
    erecv_ref[...] = jnp.sum(sharing * effmat, axis=0, keepdims=True)
    total_ref[...] = jnp.sum(sharing).reshape(1, 1)


def _run_greedy(pred, eff, pri_row, pri_col, ca_row, ca_col, n0_row, n0_col):
    full = lambda shp: pl.BlockSpec(shp, lambda: tuple(0 for _ in shp))
    args = (pred, eff, pri_row, pri_col, ca_row, ca_col, n0_row, n0_col)
    out_shape = [
        jax.ShapeDtypeStruct((_N, _N), _F32),
        jax.ShapeDtypeStruct((_N, _N), _F32),
        jax.ShapeDtypeStruct((1, _N), _F32),
        jax.ShapeDtypeStruct((_N, 1), _F32),
        jax.ShapeDtypeStruct((1, _N), _F32),
        jax.ShapeDtypeStruct((1, 1), _F32),
    ]
    return pl.pallas_call(
        _greedy_kernel,
        in_specs=[full(a.shape) for a in args],
        out_specs=[full(s.shape) for s in out_shape],
        out_shape=out_shape,
        scratch_shapes=[pltpu.VMEM((_N, _N), _F32), pltpu.VMEM((_N, _N), _F32)],
    )(*args)


def kernel(embeddings, cluster_assignments, generation, consumption,
           positions, current_hour, W1, b1, W2, b2, W3, b3,
           We1, be1, We2, be2, Wp1, bp1, Wp2, bp2):
    emb = embeddings[0].astype(_F32)
    pos = positions[0].astype(_F32)
    px = pos[:, 0:1]
    py = pos[:, 1:2]
    ca_row = cluster_assignments[0].astype(_F32).reshape(1, _N)
    ca_col = ca_row.reshape(_N, 1)
    n0_row = (generation - consumption).astype(_F32).reshape(1, _N)
    n0_col = n0_row.reshape(_N, 1)
    hour = jnp.asarray(current_hour / 24.0, _F32).reshape(1, 1)

    W1aT = W1[:, :_N].T
    W1bT = W1[:, _N:2 * _N].T
    wd_row = W1[:, 2 * _N:2 * _N + 1].reshape(1, _N)
    wh_row = W1[:, 2 * _N + 1:2 * _N + 2].reshape(1, _N)
    g_row = hour * wh_row + b1.reshape(1, -1)

    pred, eff, pri_col = _run_dense(
        emb, px, py, W1aT, W1bT, wd_row, g_row,
        W2.T, b2.reshape(1, -1), W3.T, b3.reshape(1, -1),
        We1.T, be1.reshape(1, -1), We2.T, be2.reshape(1, -1),
        Wp1.T, bp1.reshape(1, -1), Wp2.T, bp2.reshape(1, -1))

    pri_row = pri_col.reshape(1, _N)

    sharing, effmat, net, esent, erecv, total = _run_greedy(
        pred, eff, pri_row, pri_col, ca_row, ca_col, n0_row, n0_col)

    sharing3 = sharing.reshape(1, _N, _N)
    effmat3 = effmat.reshape(1, _N, _N)
    return (sharing3, effmat3, total.reshape(()), esent.reshape(1, _N),
            erecv.reshape(1, _N), net.reshape(1, _N))
